# baseline (speedup 1.0000x reference)
import functools

import jax
import jax.numpy as jnp
import numpy as np
from jax import lax
from jax.experimental import pallas as pl
from jax.experimental.pallas import tpu as pltpu
from jax.experimental.pallas import tpu_sc as plsc

_B = 32
_C = 3
_H = 512
_W = 512
_NCAMS = 32
_NIDENT = 1000
_BC = _B * _C


_ROW_IDX = np.arange(_BC, dtype=np.int32) // _C
_COL_IDX = np.arange(_BC, dtype=np.int32) % _C


def _sc_gather_body(cam_hbm, id_hbm, wcam_hbm, bcam_hbm, wident_hbm,
                    bident_hbm, rowi_hbm, coli_hbm, w_out, b_out, cam_v, id_v,
                    wcam_v, bcam_v, wident_v, bident_v, rowi_v, coli_v, w_v,
                    b_v, sem):
    wid = lax.axis_index("s") * 2 + lax.axis_index("c")

    @pl.when(wid == 0)
    def _():
        copies = [
            pltpu.async_copy(cam_hbm, cam_v, sem),
            pltpu.async_copy(id_hbm, id_v, sem),
            pltpu.async_copy(wcam_hbm, wcam_v, sem),
            pltpu.async_copy(bcam_hbm, bcam_v, sem),
            pltpu.async_copy(wident_hbm, wident_v, sem),
            pltpu.async_copy(bident_hbm, bident_v, sem),
            pltpu.async_copy(rowi_hbm, rowi_v, sem),
            pltpu.async_copy(coli_hbm, coli_v, sem),
        ]
        for c in copies:
            c.wait()
        for i in range(_BC // 16):
            row = rowi_v[pl.ds(16 * i, 16)]
            col = coli_v[pl.ds(16 * i, 16)]
            cams = plsc.load_gather(cam_v, [row])
            ids = plsc.load_gather(id_v, [row])
            cflat = cams * _C + col
            iflat = ids * _C + col
            wv = (plsc.load_gather(wcam_v, [cflat]) +
                  plsc.load_gather(wident_v, [iflat]))
            bv = (plsc.load_gather(bcam_v, [cflat]) +
                  plsc.load_gather(bident_v, [iflat]))
            w_v[pl.ds(16 * i, 16)] = wv
            b_v[pl.ds(16 * i, 16)] = bv
        outs = [pltpu.async_copy(w_v, w_out, sem),
                pltpu.async_copy(b_v, b_out, sem)]
        for c in outs:
            c.wait()


_sc_gather = functools.partial(
    pl.kernel,
    mesh=plsc.VectorSubcoreMesh(core_axis_name="c", subcore_axis_name="s"),
    compiler_params=pltpu.CompilerParams(needs_layout_passes=False),
    out_type=(jax.ShapeDtypeStruct((_BC,), jnp.float32),
              jax.ShapeDtypeStruct((_BC,), jnp.float32)),
    scratch_types=[
        pltpu.VMEM((_B,), jnp.int32),
        pltpu.VMEM((_B,), jnp.int32),
        pltpu.VMEM((_NCAMS * _C,), jnp.float32),
        pltpu.VMEM((_NCAMS * _C,), jnp.float32),
        pltpu.VMEM((_NIDENT * _C,), jnp.float32),
        pltpu.VMEM((_NIDENT * _C,), jnp.float32),
        pltpu.VMEM((_BC,), jnp.int32),
        pltpu.VMEM((_BC,), jnp.int32),
        pltpu.VMEM((_BC,), jnp.float32),
        pltpu.VMEM((_BC,), jnp.float32),
        pltpu.SemaphoreType.DMA,
    ],
)(_sc_gather_body)


_NBUF = 4
_NGROUPS = _BC // _NBUF


def _scale_bias_body(w_sm, b_sm, img_hbm, out_hbm, in_buf, out_buf, in_sem,
                     out_sem):
    for k in range(_NBUF):
        pltpu.make_async_copy(img_hbm.at[k], in_buf.at[k], in_sem.at[k]).start()

    def group(g, _):
        for k in range(_NBUF):
            i = g * _NBUF + k
            pltpu.make_async_copy(img_hbm.at[i], in_buf.at[k],
                                  in_sem.at[k]).wait()

            @pl.when(g > 0)
            def _():
                pltpu.make_async_copy(out_buf.at[k], out_hbm.at[i],
                                      out_sem.at[k]).wait()

            out_buf[k] = in_buf[k] * w_sm[i] + b_sm[i]
            pltpu.make_async_copy(out_buf.at[k], out_hbm.at[i],
                                  out_sem.at[k]).start()

            @pl.when(g < _NGROUPS - 1)
            def _():
                pltpu.make_async_copy(img_hbm.at[i + _NBUF], in_buf.at[k],
                                      in_sem.at[k]).start()
        return ()

    lax.fori_loop(0, _NGROUPS, group, ())
    for k in range(_NBUF):
        pltpu.make_async_copy(out_buf.at[k],
                              out_hbm.at[(_NGROUPS - 1) * _NBUF + k],
                              out_sem.at[k]).wait()


def kernel(image, camindex, idindex, wcam, bcam, wident, bident):
    w_flat, b_flat = _sc_gather(camindex.astype(jnp.int32),
                                idindex.astype(jnp.int32),
                                wcam.reshape(-1), bcam.reshape(-1),
                                wident.reshape(-1), bident.reshape(-1),
                                jnp.asarray(_ROW_IDX), jnp.asarray(_COL_IDX))
    img3 = image.reshape(_BC, _H, _W)
    out = pl.pallas_call(
        _scale_bias_body,
        in_specs=[
            pl.BlockSpec(memory_space=pltpu.SMEM),
            pl.BlockSpec(memory_space=pltpu.SMEM),
            pl.BlockSpec(memory_space=pl.ANY),
        ],
        out_specs=pl.BlockSpec(memory_space=pl.ANY),
        out_shape=jax.ShapeDtypeStruct((_BC, _H, _W), jnp.float32),
        scratch_shapes=[
            pltpu.VMEM((_NBUF, _H, _W), jnp.float32),
            pltpu.VMEM((_NBUF, _H, _W), jnp.float32),
            pltpu.SemaphoreType.DMA((_NBUF,)),
            pltpu.SemaphoreType.DMA((_NBUF,)),
        ],
    )(w_flat, b_flat, img3)
    return out.reshape(_B, _C, _H, _W)


# speedup vs baseline: 1.0191x; 1.0191x over previous
import functools

import jax
import jax.numpy as jnp
import numpy as np
from jax import lax
from jax.experimental import pallas as pl
from jax.experimental.pallas import tpu as pltpu
from jax.experimental.pallas import tpu_sc as plsc

_B = 32
_C = 3
_H = 512
_W = 512
_NCAMS = 32
_NIDENT = 1000
_BC = _B * _C

_OFF_BCAM = _NCAMS * _C
_OFF_WID = 2 * _NCAMS * _C
_OFF_BID = _OFF_WID + _NIDENT * _C
_PACK = _OFF_BID + _NIDENT * _C

_ROW_IDX = np.arange(_BC, dtype=np.int32) // _C
_COL_IDX = np.arange(_BC, dtype=np.int32) % _C


def _sc_gather_body(cam_hbm, id_hbm, rowi_hbm, coli_hbm, pk_hbm, wb_out,
                    cam_v, id_v, rowi_v, coli_v, pk_v, wb_v, sem):
    wid = lax.axis_index("s") * 2 + lax.axis_index("c")

    @pl.when(wid == 0)
    def _():
        copies = [
            pltpu.async_copy(cam_hbm, cam_v, sem),
            pltpu.async_copy(id_hbm, id_v, sem),
            pltpu.async_copy(rowi_hbm, rowi_v, sem),
            pltpu.async_copy(coli_hbm, coli_v, sem),
            pltpu.async_copy(pk_hbm, pk_v, sem),
        ]
        for c in copies:
            c.wait()
        for i in range(_BC // 16):
            row = rowi_v[pl.ds(16 * i, 16)]
            col = coli_v[pl.ds(16 * i, 16)]
            cams = plsc.load_gather(cam_v, [row])
            ids = plsc.load_gather(id_v, [row])
            cflat = cams * _C + col
            iflat = ids * _C + col
            wv = (plsc.load_gather(pk_v, [cflat]) +
                  plsc.load_gather(pk_v, [iflat + _OFF_WID]))
            bv = (plsc.load_gather(pk_v, [cflat + _OFF_BCAM]) +
                  plsc.load_gather(pk_v, [iflat + _OFF_BID]))
            wb_v[pl.ds(16 * i, 16)] = wv
            wb_v[pl.ds(_BC + 16 * i, 16)] = bv
        pltpu.async_copy(wb_v, wb_out, sem).wait()


_sc_gather = functools.partial(
    pl.kernel,
    mesh=plsc.VectorSubcoreMesh(core_axis_name="c", subcore_axis_name="s"),
    compiler_params=pltpu.CompilerParams(needs_layout_passes=False),
    out_type=jax.ShapeDtypeStruct((2 * _BC,), jnp.float32),
    scratch_types=[
        pltpu.VMEM((_B,), jnp.int32),
        pltpu.VMEM((_B,), jnp.int32),
        pltpu.VMEM((_BC,), jnp.int32),
        pltpu.VMEM((_BC,), jnp.int32),
        pltpu.VMEM((_PACK,), jnp.float32),
        pltpu.VMEM((2 * _BC,), jnp.float32),
        pltpu.SemaphoreType.DMA,
    ],
)(_sc_gather_body)


_NBUF = 4


def _scale_bias_body(wb_sm, img_hbm, out_hbm, in_buf, out_buf, in_sem,
                     out_sem):
    ngroups = _BC // _NBUF
    for k in range(_NBUF):
        pltpu.make_async_copy(img_hbm.at[k], in_buf.at[k], in_sem.at[k]).start()

    def group(g, _):
        for k in range(_NBUF):
            i = g * _NBUF + k
            pltpu.make_async_copy(img_hbm.at[i], in_buf.at[k],
                                  in_sem.at[k]).wait()

            @pl.when(g > 0)
            def _():
                pltpu.make_async_copy(out_buf.at[k], out_hbm.at[i],
                                      out_sem.at[k]).wait()

            out_buf[k] = in_buf[k] * wb_sm[i] + wb_sm[_BC + i]
            pltpu.make_async_copy(out_buf.at[k], out_hbm.at[i],
                                  out_sem.at[k]).start()

            @pl.when(g < ngroups - 1)
            def _():
                pltpu.make_async_copy(img_hbm.at[i + _NBUF], in_buf.at[k],
                                      in_sem.at[k]).start()
        return ()

    lax.fori_loop(0, ngroups, group, ())
    for k in range(_NBUF):
        pltpu.make_async_copy(out_buf.at[k],
                              out_hbm.at[(ngroups - 1) * _NBUF + k],
                              out_sem.at[k]).wait()


def kernel(image, camindex, idindex, wcam, bcam, wident, bident):
    pack = jnp.concatenate([
        wcam.reshape(-1), bcam.reshape(-1),
        wident.reshape(-1), bident.reshape(-1)
    ])
    wb = _sc_gather(camindex.astype(jnp.int32), idindex.astype(jnp.int32),
                    jnp.asarray(_ROW_IDX), jnp.asarray(_COL_IDX), pack)
    img3 = image.reshape(_BC, _H, _W)
    out = pl.pallas_call(
        _scale_bias_body,
        in_specs=[
            pl.BlockSpec(memory_space=pltpu.SMEM),
            pl.BlockSpec(memory_space=pl.ANY),
        ],
        out_specs=pl.BlockSpec(memory_space=pl.ANY),
        out_shape=jax.ShapeDtypeStruct((_BC, _H, _W), jnp.float32),
        scratch_shapes=[
            pltpu.VMEM((_NBUF, _H, _W), jnp.float32),
            pltpu.VMEM((_NBUF, _H, _W), jnp.float32),
            pltpu.SemaphoreType.DMA((_NBUF,)),
            pltpu.SemaphoreType.DMA((_NBUF,)),
        ],
    )(wb, img3)
    return out.reshape(_B, _C, _H, _W)


# speedup vs baseline: 1.0193x; 1.0002x over previous
import functools

import jax
import jax.numpy as jnp
import numpy as np
from jax import lax
from jax.experimental import pallas as pl
from jax.experimental.pallas import tpu as pltpu
from jax.experimental.pallas import tpu_sc as plsc

_B = 32
_C = 3
_H = 512
_W = 512
_NCAMS = 32
_NIDENT = 1000
_BC = _B * _C

def _sc_gather_body(cam_hbm, id_hbm, wcam_hbm, bcam_hbm, wident_hbm,
                    bident_hbm, wb_out, cam_v, id_v, wcam_v, bcam_v, wid_v,
                    bid_v, wb_v, sem):
    wid = lax.axis_index("s") * 2 + lax.axis_index("c")

    @pl.when(wid == 0)
    def _():
        idx = [
            pltpu.async_copy(cam_hbm, cam_v, sem),
            pltpu.async_copy(id_hbm, id_v, sem),
        ]
        for c in idx:
            c.wait()
        copies = [
            pltpu.async_copy(wcam_hbm, wcam_v, sem),
            pltpu.async_copy(bcam_hbm, bcam_v, sem),
            pltpu.async_copy(wident_hbm, wid_v, sem),
            pltpu.async_copy(bident_hbm, bid_v, sem),
        ]
        for c in copies:
            c.wait()
        lanes = lax.broadcasted_iota(jnp.int32, (16,), 0)
        for i in range(_BC // 16):
            flat = lanes + (16 * i)
            row = lax.shift_right_logical(flat * 21846, 16)
            col = flat - row * _C
            cams = plsc.load_gather(cam_v, [row])
            ids = plsc.load_gather(id_v, [row])
            iflat = ids * _C + col
            wv = (plsc.load_gather(wcam_v, [cams, col]) +
                  plsc.load_gather(wid_v, [iflat]))
            bv = (plsc.load_gather(bcam_v, [cams, col]) +
                  plsc.load_gather(bid_v, [iflat]))
            wb_v[pl.ds(16 * i, 16)] = wv
            wb_v[pl.ds(_BC + 16 * i, 16)] = bv
        pltpu.async_copy(wb_v, wb_out, sem).wait()


_sc_gather = functools.partial(
    pl.kernel,
    mesh=plsc.VectorSubcoreMesh(core_axis_name="c", subcore_axis_name="s"),
    compiler_params=pltpu.CompilerParams(needs_layout_passes=False),
    out_type=jax.ShapeDtypeStruct((2 * _BC,), jnp.float32),
    scratch_types=[
        pltpu.VMEM((_B,), jnp.int32),
        pltpu.VMEM((_B,), jnp.int32),
        pltpu.VMEM((_NCAMS, _C), jnp.float32),
        pltpu.VMEM((_NCAMS, _C), jnp.float32),
        pltpu.VMEM((_NIDENT * _C,), jnp.float32),
        pltpu.VMEM((_NIDENT * _C,), jnp.float32),
        pltpu.VMEM((2 * _BC,), jnp.float32),
        pltpu.SemaphoreType.DMA,
    ],
)(_sc_gather_body)


_NBUF = 4


def _scale_bias_body(wb_sm, img_hbm, out_hbm, in_buf, out_buf, in_sem,
                     out_sem):
    ngroups = _BC // _NBUF
    for k in range(_NBUF):
        pltpu.make_async_copy(img_hbm.at[k], in_buf.at[k], in_sem.at[k]).start()

    def group(g, _):
        for k in range(_NBUF):
            i = g * _NBUF + k
            pltpu.make_async_copy(img_hbm.at[i], in_buf.at[k],
                                  in_sem.at[k]).wait()

            @pl.when(g > 0)
            def _():
                pltpu.make_async_copy(out_buf.at[k], out_hbm.at[i],
                                      out_sem.at[k]).wait()

            out_buf[k] = in_buf[k] * wb_sm[i] + wb_sm[_BC + i]
            pltpu.make_async_copy(out_buf.at[k], out_hbm.at[i],
                                  out_sem.at[k]).start()

            @pl.when(g < ngroups - 1)
            def _():
                pltpu.make_async_copy(img_hbm.at[i + _NBUF], in_buf.at[k],
                                      in_sem.at[k]).start()
        return ()

    lax.fori_loop(0, ngroups, group, ())
    for k in range(_NBUF):
        pltpu.make_async_copy(out_buf.at[k],
                              out_hbm.at[(ngroups - 1) * _NBUF + k],
                              out_sem.at[k]).wait()


def kernel(image, camindex, idindex, wcam, bcam, wident, bident):
    wb = _sc_gather(camindex.astype(jnp.int32), idindex.astype(jnp.int32),
                    wcam, bcam, wident.reshape(-1), bident.reshape(-1))
    img3 = image.reshape(_BC, _H, _W)
    out = pl.pallas_call(
        _scale_bias_body,
        in_specs=[
            pl.BlockSpec(memory_space=pltpu.SMEM),
            pl.BlockSpec(memory_space=pl.ANY),
        ],
        out_specs=pl.BlockSpec(memory_space=pl.ANY),
        out_shape=jax.ShapeDtypeStruct((_BC, _H, _W), jnp.float32),
        scratch_shapes=[
            pltpu.VMEM((_NBUF, _H, _W), jnp.float32),
            pltpu.VMEM((_NBUF, _H, _W), jnp.float32),
            pltpu.SemaphoreType.DMA((_NBUF,)),
            pltpu.SemaphoreType.DMA((_NBUF,)),
        ],
    )(wb, img3)
    return out.reshape(_B, _C, _H, _W)


# speedup vs baseline: 1.0364x; 1.0168x over previous
import functools

import jax
import jax.numpy as jnp
import numpy as np
from jax import lax
from jax.experimental import pallas as pl
from jax.experimental.pallas import tpu as pltpu
from jax.experimental.pallas import tpu_sc as plsc

_B = 32
_C = 3
_H = 512
_W = 512
_NCAMS = 32
_NIDENT = 1000
_BC = _B * _C

def _sc_gather_body(cam_hbm, id_hbm, wcam_hbm, bcam_hbm, wident_hbm,
                    bident_hbm, wb_out, cam_v, id_v, wcam_v, bcam_v, wid_v,
                    bid_v, wb_v, sem):
    wid = lax.axis_index("s") * 2 + lax.axis_index("c")

    @pl.when(wid == 0)
    def _():
        idx = [
            pltpu.async_copy(cam_hbm, cam_v, sem),
            pltpu.async_copy(id_hbm, id_v, sem),
        ]
        for c in idx:
            c.wait()
        copies = [
            pltpu.async_copy(wcam_hbm, wcam_v, sem),
            pltpu.async_copy(bcam_hbm, bcam_v, sem),
            pltpu.async_copy(wident_hbm, wid_v, sem),
            pltpu.async_copy(bident_hbm, bid_v, sem),
        ]
        for c in copies:
            c.wait()
        lanes = lax.broadcasted_iota(jnp.int32, (16,), 0)
        for i in range(_BC // 16):
            flat = lanes + (16 * i)
            row = lax.shift_right_logical(flat * 21846, 16)
            col = flat - row * _C
            cams = plsc.load_gather(cam_v, [row])
            ids = plsc.load_gather(id_v, [row])
            iflat = ids * _C + col
            wv = (plsc.load_gather(wcam_v, [cams, col]) +
                  plsc.load_gather(wid_v, [iflat]))
            bv = (plsc.load_gather(bcam_v, [cams, col]) +
                  plsc.load_gather(bid_v, [iflat]))
            wb_v[pl.ds(16 * i, 16)] = wv
            wb_v[pl.ds(_BC + 16 * i, 16)] = bv
        pltpu.async_copy(wb_v, wb_out, sem).wait()


_sc_gather = functools.partial(
    pl.kernel,
    mesh=plsc.VectorSubcoreMesh(core_axis_name="c", subcore_axis_name="s",
                                num_cores=1),
    compiler_params=pltpu.CompilerParams(needs_layout_passes=False),
    out_type=jax.ShapeDtypeStruct((2 * _BC,), jnp.float32),
    scratch_types=[
        pltpu.VMEM((_B,), jnp.int32),
        pltpu.VMEM((_B,), jnp.int32),
        pltpu.VMEM((_NCAMS, _C), jnp.float32),
        pltpu.VMEM((_NCAMS, _C), jnp.float32),
        pltpu.VMEM((_NIDENT * _C,), jnp.float32),
        pltpu.VMEM((_NIDENT * _C,), jnp.float32),
        pltpu.VMEM((2 * _BC,), jnp.float32),
        pltpu.SemaphoreType.DMA,
    ],
)(_sc_gather_body)


_NBUF = 4


def _scale_bias_body(wb_sm, img_hbm, out_hbm, in_buf, out_buf, in_sem,
                     out_sem):
    ngroups = _BC // _NBUF
    for k in range(_NBUF):
        pltpu.make_async_copy(img_hbm.at[k], in_buf.at[k], in_sem.at[k]).start()

    def group(g, _):
        for k in range(_NBUF):
            i = g * _NBUF + k
            pltpu.make_async_copy(img_hbm.at[i], in_buf.at[k],
                                  in_sem.at[k]).wait()

            @pl.when(g > 0)
            def _():
                pltpu.make_async_copy(out_buf.at[k], out_hbm.at[i],
                                      out_sem.at[k]).wait()

            out_buf[k] = in_buf[k] * wb_sm[i] + wb_sm[_BC + i]
            pltpu.make_async_copy(out_buf.at[k], out_hbm.at[i],
                                  out_sem.at[k]).start()

            @pl.when(g < ngroups - 1)
            def _():
                pltpu.make_async_copy(img_hbm.at[i + _NBUF], in_buf.at[k],
                                      in_sem.at[k]).start()
        return ()

    lax.fori_loop(0, ngroups, group, ())
    for k in range(_NBUF):
        pltpu.make_async_copy(out_buf.at[k],
                              out_hbm.at[(ngroups - 1) * _NBUF + k],
                              out_sem.at[k]).wait()


def kernel(image, camindex, idindex, wcam, bcam, wident, bident):
    wb = _sc_gather(camindex.astype(jnp.int32), idindex.astype(jnp.int32),
                    wcam, bcam, wident.reshape(-1), bident.reshape(-1))
    img3 = image.reshape(_BC, _H, _W)
    out = pl.pallas_call(
        _scale_bias_body,
        in_specs=[
            pl.BlockSpec(memory_space=pltpu.SMEM),
            pl.BlockSpec(memory_space=pl.ANY),
        ],
        out_specs=pl.BlockSpec(memory_space=pl.ANY),
        out_shape=jax.ShapeDtypeStruct((_BC, _H, _W), jnp.float32),
        scratch_shapes=[
            pltpu.VMEM((_NBUF, _H, _W), jnp.float32),
            pltpu.VMEM((_NBUF, _H, _W), jnp.float32),
            pltpu.SemaphoreType.DMA((_NBUF,)),
            pltpu.SemaphoreType.DMA((_NBUF,)),
        ],
    )(wb, img3)
    return out.reshape(_B, _C, _H, _W)


# speedup vs baseline: 1.0586x; 1.0214x over previous
import functools

import jax
import jax.numpy as jnp
import numpy as np
from jax import lax
from jax.experimental import pallas as pl
from jax.experimental.pallas import tpu as pltpu
from jax.experimental.pallas import tpu_sc as plsc

_B = 32
_C = 3
_H = 512
_W = 512
_NCAMS = 32
_NIDENT = 1000
_BC = _B * _C

def _sc_gather_body(cam_hbm, id_hbm, wcam_hbm, bcam_hbm, wident_hbm,
                    bident_hbm, wb_out, cam_v, id_v, wcam_v, bcam_v, wid_v,
                    bid_v, wb_v, sem):
    wid = lax.axis_index("s") * 2 + lax.axis_index("c")

    @pl.when(wid == 0)
    def _():
        idx = [
            pltpu.async_copy(cam_hbm, cam_v, sem),
            pltpu.async_copy(id_hbm, id_v, sem),
        ]
        for c in idx:
            c.wait()
        copies = [
            pltpu.async_copy(wcam_hbm, wcam_v, sem),
            pltpu.async_copy(bcam_hbm, bcam_v, sem),
            pltpu.async_copy(wident_hbm, wid_v, sem),
            pltpu.async_copy(bident_hbm, bid_v, sem),
        ]
        for c in copies:
            c.wait()
        lanes = lax.broadcasted_iota(jnp.int32, (16,), 0)
        for i in range(_BC // 16):
            flat = lanes + (16 * i)
            row = lax.shift_right_logical(flat * 21846, 16)
            col = flat - row * _C
            cams = plsc.load_gather(cam_v, [row])
            ids = plsc.load_gather(id_v, [row])
            iflat = ids * _C + col
            wv = (plsc.load_gather(wcam_v, [cams, col]) +
                  plsc.load_gather(wid_v, [iflat]))
            bv = (plsc.load_gather(bcam_v, [cams, col]) +
                  plsc.load_gather(bid_v, [iflat]))
            wb_v[pl.ds(16 * i, 16)] = wv
            wb_v[pl.ds(_BC + 16 * i, 16)] = bv
        pltpu.async_copy(wb_v, wb_out, sem).wait()


_sc_gather = functools.partial(
    pl.kernel,
    mesh=plsc.VectorSubcoreMesh(core_axis_name="c", subcore_axis_name="s",
                                num_cores=1),
    compiler_params=pltpu.CompilerParams(needs_layout_passes=False),
    out_type=jax.ShapeDtypeStruct((2 * _BC,), jnp.float32),
    scratch_types=[
        pltpu.VMEM((_B,), jnp.int32),
        pltpu.VMEM((_B,), jnp.int32),
        pltpu.VMEM((_NCAMS, _C), jnp.float32),
        pltpu.VMEM((_NCAMS, _C), jnp.float32),
        pltpu.VMEM((_NIDENT * _C,), jnp.float32),
        pltpu.VMEM((_NIDENT * _C,), jnp.float32),
        pltpu.VMEM((2 * _BC,), jnp.float32),
        pltpu.SemaphoreType.DMA,
    ],
)(_sc_gather_body)


_NBUF = 4


_RPC = 2
_NCHUNK = _BC // _RPC


def _scale_bias_body(wb_sm, img_hbm, out_hbm, in_buf, out_buf, in_sem,
                     out_sem):
    ngroups = _NCHUNK // _NBUF

    def chunk(i):
        return pl.ds(i * _RPC, _RPC)

    for k in range(_NBUF):
        pltpu.make_async_copy(img_hbm.at[chunk(k)], in_buf.at[k],
                              in_sem.at[k]).start()

    def group(g, _):
        for k in range(_NBUF):
            i = g * _NBUF + k
            pltpu.make_async_copy(img_hbm.at[chunk(i)], in_buf.at[k],
                                  in_sem.at[k]).wait()

            @pl.when(g > 0)
            def _():
                pltpu.make_async_copy(out_buf.at[k], out_hbm.at[chunk(i)],
                                      out_sem.at[k]).wait()

            for r in range(_RPC):
                out_buf[k, r] = (in_buf[k, r] * wb_sm[i * _RPC + r] +
                                 wb_sm[_BC + i * _RPC + r])
            pltpu.make_async_copy(out_buf.at[k], out_hbm.at[chunk(i)],
                                  out_sem.at[k]).start()

            @pl.when(g < ngroups - 1)
            def _():
                pltpu.make_async_copy(img_hbm.at[chunk(i + _NBUF)],
                                      in_buf.at[k], in_sem.at[k]).start()
        return ()

    lax.fori_loop(0, ngroups, group, ())
    for k in range(_NBUF):
        pltpu.make_async_copy(out_buf.at[k],
                              out_hbm.at[chunk((ngroups - 1) * _NBUF + k)],
                              out_sem.at[k]).wait()


def kernel(image, camindex, idindex, wcam, bcam, wident, bident):
    wb = _sc_gather(camindex.astype(jnp.int32), idindex.astype(jnp.int32),
                    wcam, bcam, wident.reshape(-1), bident.reshape(-1))
    img3 = image.reshape(_BC, _H, _W)
    out = pl.pallas_call(
        _scale_bias_body,
        in_specs=[
            pl.BlockSpec(memory_space=pltpu.SMEM),
            pl.BlockSpec(memory_space=pl.ANY),
        ],
        out_specs=pl.BlockSpec(memory_space=pl.ANY),
        out_shape=jax.ShapeDtypeStruct((_BC, _H, _W), jnp.float32),
        scratch_shapes=[
            pltpu.VMEM((_NBUF, _RPC, _H, _W), jnp.float32),
            pltpu.VMEM((_NBUF, _RPC, _H, _W), jnp.float32),
            pltpu.SemaphoreType.DMA((_NBUF,)),
            pltpu.SemaphoreType.DMA((_NBUF,)),
        ],
    )(wb, img3)
    return out.reshape(_B, _C, _H, _W)


# speedup vs baseline: 1.0630x; 1.0041x over previous
import functools

import jax
import jax.numpy as jnp
import numpy as np
from jax import lax
from jax.experimental import pallas as pl
from jax.experimental.pallas import tpu as pltpu
from jax.experimental.pallas import tpu_sc as plsc

_B = 32
_C = 3
_H = 512
_W = 512
_NCAMS = 32
_NIDENT = 1000
_BC = _B * _C

def _sc_gather_body(cam_hbm, id_hbm, wcam_hbm, bcam_hbm, wident_hbm,
                    bident_hbm, wb_out, cam_v, id_v, wcam_v, bcam_v, wid_v,
                    bid_v, wb_v, sem):
    wid = lax.axis_index("s") * 2 + lax.axis_index("c")

    @pl.when(wid == 0)
    def _():
        idx = [
            pltpu.async_copy(cam_hbm, cam_v, sem),
            pltpu.async_copy(id_hbm, id_v, sem),
        ]
        for c in idx:
            c.wait()
        copies = [
            pltpu.async_copy(wcam_hbm, wcam_v, sem),
            pltpu.async_copy(bcam_hbm, bcam_v, sem),
            pltpu.async_copy(wident_hbm, wid_v, sem),
            pltpu.async_copy(bident_hbm, bid_v, sem),
        ]
        for c in copies:
            c.wait()
        lanes = lax.broadcasted_iota(jnp.int32, (16,), 0)
        for i in range(_BC // 16):
            flat = lanes + (16 * i)
            row = lax.shift_right_logical(flat * 21846, 16)
            col = flat - row * _C
            cams = plsc.load_gather(cam_v, [row])
            ids = plsc.load_gather(id_v, [row])
            iflat = ids * _C + col
            wv = (plsc.load_gather(wcam_v, [cams, col]) +
                  plsc.load_gather(wid_v, [iflat]))
            bv = (plsc.load_gather(bcam_v, [cams, col]) +
                  plsc.load_gather(bid_v, [iflat]))
            wb_v[pl.ds(16 * i, 16)] = wv
            wb_v[pl.ds(_BC + 16 * i, 16)] = bv
        pltpu.async_copy(wb_v, wb_out, sem).wait()


_sc_gather = functools.partial(
    pl.kernel,
    mesh=plsc.VectorSubcoreMesh(core_axis_name="c", subcore_axis_name="s",
                                num_cores=1),
    compiler_params=pltpu.CompilerParams(needs_layout_passes=False),
    out_type=jax.ShapeDtypeStruct((2 * _BC,), jnp.float32),
    scratch_types=[
        pltpu.VMEM((_B,), jnp.int32),
        pltpu.VMEM((_B,), jnp.int32),
        pltpu.VMEM((_NCAMS, _C), jnp.float32),
        pltpu.VMEM((_NCAMS, _C), jnp.float32),
        pltpu.VMEM((_NIDENT * _C,), jnp.float32),
        pltpu.VMEM((_NIDENT * _C,), jnp.float32),
        pltpu.VMEM((2 * _BC,), jnp.float32),
        pltpu.SemaphoreType.DMA,
    ],
)(_sc_gather_body)


_NBUF = 4


_RPC = 4
_NCHUNK = _BC // _RPC


def _scale_bias_body(wb_sm, img_hbm, out_hbm, in_buf, out_buf, in_sem,
                     out_sem):
    ngroups = _NCHUNK // _NBUF

    def chunk(i):
        return pl.ds(i * _RPC, _RPC)

    for k in range(_NBUF):
        pltpu.make_async_copy(img_hbm.at[chunk(k)], in_buf.at[k],
                              in_sem.at[k]).start()

    def group(g, _):
        for k in range(_NBUF):
            i = g * _NBUF + k
            pltpu.make_async_copy(img_hbm.at[chunk(i)], in_buf.at[k],
                                  in_sem.at[k]).wait()

            @pl.when(g > 0)
            def _():
                pltpu.make_async_copy(out_buf.at[k], out_hbm.at[chunk(i)],
                                      out_sem.at[k]).wait()

            for r in range(_RPC):
                out_buf[k, r] = (in_buf[k, r] * wb_sm[i * _RPC + r] +
                                 wb_sm[_BC + i * _RPC + r])
            pltpu.make_async_copy(out_buf.at[k], out_hbm.at[chunk(i)],
                                  out_sem.at[k]).start()

            @pl.when(g < ngroups - 1)
            def _():
                pltpu.make_async_copy(img_hbm.at[chunk(i + _NBUF)],
                                      in_buf.at[k], in_sem.at[k]).start()
        return ()

    lax.fori_loop(0, ngroups, group, ())
    for k in range(_NBUF):
        pltpu.make_async_copy(out_buf.at[k],
                              out_hbm.at[chunk((ngroups - 1) * _NBUF + k)],
                              out_sem.at[k]).wait()


def kernel(image, camindex, idindex, wcam, bcam, wident, bident):
    wb = _sc_gather(camindex.astype(jnp.int32), idindex.astype(jnp.int32),
                    wcam, bcam, wident.reshape(-1), bident.reshape(-1))
    img3 = image.reshape(_BC, _H, _W)
    out = pl.pallas_call(
        _scale_bias_body,
        in_specs=[
            pl.BlockSpec(memory_space=pltpu.SMEM),
            pl.BlockSpec(memory_space=pl.ANY),
        ],
        out_specs=pl.BlockSpec(memory_space=pl.ANY),
        out_shape=jax.ShapeDtypeStruct((_BC, _H, _W), jnp.float32),
        scratch_shapes=[
            pltpu.VMEM((_NBUF, _RPC, _H, _W), jnp.float32),
            pltpu.VMEM((_NBUF, _RPC, _H, _W), jnp.float32),
            pltpu.SemaphoreType.DMA((_NBUF,)),
            pltpu.SemaphoreType.DMA((_NBUF,)),
        ],
    )(wb, img3)
    return out.reshape(_B, _C, _H, _W)
